# baseline (speedup 1.0000x reference)
import jax
import jax.numpy as jnp
from jax import lax
from jax.experimental import pallas as pl
from jax.experimental.pallas import tpu as pltpu
from jax.experimental.pallas import tpu_sc as plsc

_NEQ = 10000
_B = 64
_XD = 12000
_HID = 512
_YD = 1000
_NE = 160000

_NS = 16
_NPAD = 10240
_NPT = _NPAD // _NS
_EPT = _NE // _NS
_CC = 157
_CORNER = _CC * _B
_CPAD = 160
_XCN = 256

_BK = 2048
_NBLK = 6


def _rsqrt16(x):
    i = plsc.bitcast(x, jnp.int32)
    i = jnp.int32(0x5F3759DF) - lax.shift_right_logical(i, 1)
    y = plsc.bitcast(i, jnp.float32)
    for _ in range(3):
        y = y * (1.5 - 0.5 * x * y * y)
    return y


def _sc_body(rows_hbm, cols_hbm, x_hbm, out_hbm,
             rows_v, cols_v, xcn_v, xd_v, cnt_loc, s_loc, m_v, m_full,
             dinv_v, dsqp_v, ep_v, xdp_v, red_v, sh_all, m_sh):
    s = lax.axis_index("s")
    c = lax.axis_index("c")
    eoff = s * _EPT
    noff = s * _NPT
    zero16 = jnp.zeros((16,), jnp.float32)
    one16 = jnp.ones((16,), jnp.float32)
    lane16 = lax.iota(jnp.int32, 16)
    zero16i = jnp.zeros((16,), jnp.int32)

    @pl.when(c == 0)
    def _phase1():
        pltpu.sync_copy(rows_hbm.at[pl.ds(eoff, _EPT)], rows_v)
        pltpu.sync_copy(cols_hbm.at[pl.ds(eoff, _EPT)], cols_v)
        pltpu.sync_copy(x_hbm.at[:, pl.ds(0, _XCN)], xcn_v)

        @plsc.parallel_loop(0, _NPAD // 16, unroll=8)
        def _zero_body(i):
            cnt_loc[pl.ds(i * 16, 16)] = zero16
            s_loc[pl.ds(i * 16, 16)] = zero16

        @plsc.parallel_loop(0, _EPT // 16, unroll=5)
        def _hist_body(i):
            idx = cols_v[pl.ds(i * 16, 16)]
            plsc.addupdate_scatter(cnt_loc, [idx], one16)

        pltpu.sync_copy(cnt_loc, sh_all.at[s])

    plsc.subcore_barrier()

    @pl.when(c == 0)
    def _phase2():
        pltpu.sync_copy(sh_all.at[:, pl.ds(noff, _NPT)], red_v)

        @plsc.parallel_loop(0, _NPT // 16, unroll=4)
        def _dinv_body(j):
            o = j * 16
            tot = red_v[0, pl.ds(o, 16)]
            for t in range(1, _NS):
                tot = tot + red_v[t, pl.ds(o, 16)]
            deg = tot * 64.0 + 1.0
            y = _rsqrt16(deg)
            bidx = lane16 + (o % 64)
            nidx = zero16i + (s * (_NPT // 64) + o // 64)
            xd = plsc.load_gather(xcn_v, [bidx, nidx])
            po = (o // 64) * 128 + (o % 64)
            xd_v[pl.ds(o, 16)] = xd
            xdp_v[pl.ds(po, 16)] = xd
            dinv_v[pl.ds(o, 16)] = y
            dsqp_v[pl.ds(po, 16)] = y * y
            m_v[pl.ds(o, 16)] = xd * y

        pltpu.sync_copy(m_v, m_sh.at[pl.ds(noff, _NPT)])

    plsc.subcore_barrier()

    @pl.when(c == 0)
    def _phase3():
        pltpu.sync_copy(m_sh, m_full)

        @plsc.parallel_loop(0, _EPT // 16, unroll=5)
        def _msg_body(i):
            o = i * 16
            ridx = rows_v[pl.ds(o, 16)]
            cidx = cols_v[pl.ds(o, 16)]
            mv = plsc.load_gather(m_full, [ridx])
            plsc.addupdate_scatter(s_loc, [cidx], mv)

        pltpu.sync_copy(s_loc, sh_all.at[s])

    plsc.subcore_barrier()

    @pl.when(c == 0)
    def _phase4():
        pltpu.sync_copy(sh_all.at[:, pl.ds(noff, _NPT)], red_v)

        @plsc.parallel_loop(0, _NPT // 16, unroll=4)
        def _e_body(j):
            o = j * 16
            tot = red_v[0, pl.ds(o, 16)]
            for t in range(1, _NS):
                tot = tot + red_v[t, pl.ds(o, 16)]
            po = (o // 64) * 128 + (o % 64)
            ep_v[pl.ds(po, 16)] = 64.0 * dinv_v[pl.ds(o, 16)] * tot

        poff = s * (_NPT * 2)
        pltpu.sync_copy(dsqp_v, out_hbm.at[pl.ds(poff, _NPT * 2)])
        pltpu.sync_copy(ep_v, out_hbm.at[pl.ds(_NPAD * 2 + poff, _NPT * 2)])
        pltpu.sync_copy(xdp_v, out_hbm.at[pl.ds(_NPAD * 4 + poff, _NPT * 2)])


_sc_edge = pl.kernel(
    _sc_body,
    out_type=jax.ShapeDtypeStruct((6 * _NPAD,), jnp.float32),
    mesh=plsc.VectorSubcoreMesh(core_axis_name="c", subcore_axis_name="s"),
    compiler_params=pltpu.CompilerParams(needs_layout_passes=False),
    scratch_types=[
        pltpu.VMEM((_EPT,), jnp.int32),
        pltpu.VMEM((_EPT,), jnp.int32),
        pltpu.VMEM((_B, _XCN), jnp.float32),
        pltpu.VMEM((_NPT,), jnp.float32),
        pltpu.VMEM((_NPAD,), jnp.float32),
        pltpu.VMEM((_NPAD,), jnp.float32),
        pltpu.VMEM((_NPT,), jnp.float32),
        pltpu.VMEM((_NPAD,), jnp.float32),
        pltpu.VMEM((_NPT,), jnp.float32),
        pltpu.VMEM((2 * _NPT,), jnp.float32),
        pltpu.VMEM((2 * _NPT,), jnp.float32),
        pltpu.VMEM((2 * _NPT,), jnp.float32),
        pltpu.VMEM((_NS, _NPT), jnp.float32),
        pltpu.VMEM_SHARED((_NS, _NPAD), jnp.float32),
        pltpu.VMEM_SHARED((_NPAD,), jnp.float32),
    ],
)


def _tc_main_body(gp_ref, x_ref, w1_ref, o_ref):
    k = pl.program_id(0)
    gw = gp_ref[0]
    gb = gp_ref[1]
    xb = x_ref[...]
    hb = jnp.maximum(gw * xb + gb, 0.0)
    col = k * _BK + lax.broadcasted_iota(jnp.int32, (_B, _BK), 1)
    x2b = jnp.where(col < _NEQ, hb, jnp.where(col < _XD, xb, 0.0))
    part = lax.dot_general(x2b, w1_ref[...], (((1,), (0,)), ((), ())),
                           preferred_element_type=jnp.float32)

    @pl.when(k == 0)
    def _():
        o_ref[...] = part

    @pl.when(k != 0)
    def _():
        o_ref[...] = o_ref[...] + part


_tc_main = pl.pallas_call(
    _tc_main_body,
    grid=(_NBLK,),
    in_specs=[
        pl.BlockSpec(memory_space=pltpu.SMEM),
        pl.BlockSpec((_B, _BK), lambda k: (0, k)),
        pl.BlockSpec((_BK, _HID), lambda k: (k, 0)),
    ],
    out_specs=pl.BlockSpec((_B, _HID), lambda k: (0, 0)),
    out_shape=jax.ShapeDtypeStruct((_B, _HID), jnp.float32),
    compiler_params=pltpu.CompilerParams(dimension_semantics=("arbitrary",)),
)


def _tc_fix_body(gp_ref, sc_ref, w1_ref, acc_ref, b1_ref, w2_ref,
                 b2_ref, o_ref):
    gw = gp_ref[0]
    gb = gp_ref[1]
    a = sc_ref[0]
    cc = sc_ref[1]
    xt = sc_ref[2]
    h_plain = jnp.maximum(gw * xt + gb, 0.0)
    h_corner = jnp.maximum(gw * (xt * a + cc) + gb, 0.0)
    delta_t = h_corner - h_plain
    fix = lax.dot_general(delta_t, w1_ref[:_CPAD, :], (((0,), (0,)), ((), ())),
                          preferred_element_type=jnp.float32)[:_B, :]
    h1 = jnp.maximum(acc_ref[...] + fix + b1_ref[...], 0.0)
    o_ref[...] = lax.dot_general(h1, w2_ref[...], (((1,), (0,)), ((), ())),
                                 preferred_element_type=jnp.float32) + b2_ref[...]


_tc_fix = pl.pallas_call(
    _tc_fix_body,
    grid=(1,),
    in_specs=[
        pl.BlockSpec(memory_space=pltpu.SMEM),
        pl.BlockSpec((3, _CPAD, 128), lambda k: (0, 0, 0)),
        pl.BlockSpec((_CPAD, _HID), lambda k: (0, 0)),
        pl.BlockSpec((_B, _HID), lambda k: (0, 0)),
        pl.BlockSpec((1, _HID), lambda k: (0, 0)),
        pl.BlockSpec((_HID, _YD), lambda k: (0, 0)),
        pl.BlockSpec((1, _YD), lambda k: (0, 0)),
    ],
    out_specs=pl.BlockSpec((_B, _YD), lambda k: (0, 0)),
    out_shape=jax.ShapeDtypeStruct((_B, _YD), jnp.float32),
)


def kernel(x, edge_index, gcn_w, gcn_b, w1, b1, w2, b2):
    gp = jnp.concatenate([gcn_w.reshape(-1), gcn_b.reshape(-1)])
    sc_out = _sc_edge(edge_index[0], edge_index[1], x)
    acc = _tc_main(gp, x, w1)
    sc3 = sc_out.reshape(3, _CPAD, 128)
    return _tc_fix(gp, sc3, w1, acc, b1.reshape(1, _HID), w2,
                   b2.reshape(1, _YD))


# speedup vs baseline: 1.1280x; 1.1280x over previous
import jax
import jax.numpy as jnp
from jax import lax
from jax.experimental import pallas as pl
from jax.experimental.pallas import tpu as pltpu
from jax.experimental.pallas import tpu_sc as plsc

_NEQ = 10000
_B = 64
_XD = 12000
_HID = 512
_YD = 1000
_NE = 160000

_NS = 16
_NPAD = 10240
_NPT = _NPAD // _NS
_EPT = _NE // _NS
_CC = 157
_CORNER = _CC * _B
_CPAD = 160
_XCN = 256
_EPTW = 10112

_BK = 2048
_NBLK = 6


def _rsqrt16(x):
    i = plsc.bitcast(x, jnp.int32)
    i = jnp.int32(0x5F3759DF) - lax.shift_right_logical(i, 1)
    y = plsc.bitcast(i, jnp.float32)
    for _ in range(3):
        y = y * (1.5 - 0.5 * x * y * y)
    return y


def _sc_body(ei_hbm, x_hbm, out_hbm,
             ei_v, xcn_v, xd_v, cnt_loc, s_loc, m_v, m_full,
             dinv_v, dsqp_v, ep_v, xdp_v, red_v, sh_all, m_sh):
    s = lax.axis_index("s")
    c = lax.axis_index("c")
    eoff = s * _EPT
    noff = s * _NPT
    zero16 = jnp.zeros((16,), jnp.float32)
    one16 = jnp.ones((16,), jnp.float32)
    lane16 = lax.iota(jnp.int32, 16)
    zero16i = jnp.zeros((16,), jnp.int32)

    estart = pl.multiple_of((eoff // 128) * 128, 128)
    delta = eoff - estart

    @pl.when(c == 0)
    def _phase1():
        pltpu.sync_copy(ei_hbm.at[:, pl.ds(estart, _EPTW)], ei_v)
        pltpu.sync_copy(x_hbm.at[:, pl.ds(0, _XCN)], xcn_v)

        @plsc.parallel_loop(0, _NPAD // 16, unroll=8)
        def _zero_body(i):
            cnt_loc[pl.ds(i * 16, 16)] = zero16
            s_loc[pl.ds(i * 16, 16)] = zero16

        @plsc.parallel_loop(0, _EPT // 16, unroll=5)
        def _hist_body(i):
            idx = ei_v[1, pl.ds(delta + i * 16, 16)]
            plsc.addupdate_scatter(cnt_loc, [idx], one16)

        pltpu.sync_copy(cnt_loc, sh_all.at[s])

    plsc.subcore_barrier()

    @pl.when(c == 0)
    def _phase2():
        pltpu.sync_copy(sh_all.at[:, pl.ds(noff, _NPT)], red_v)

        @plsc.parallel_loop(0, _NPT // 16, unroll=4)
        def _dinv_body(j):
            o = j * 16
            tot = red_v[0, pl.ds(o, 16)]
            for t in range(1, _NS):
                tot = tot + red_v[t, pl.ds(o, 16)]
            deg = tot * 64.0 + 1.0
            y = _rsqrt16(deg)
            bidx = lane16 + (o % 64)
            nidx = zero16i + (s * (_NPT // 64) + o // 64)
            xd = plsc.load_gather(xcn_v, [bidx, nidx])
            po = (o // 64) * 128 + (o % 64)
            xd_v[pl.ds(o, 16)] = xd
            xdp_v[pl.ds(po, 16)] = xd
            dinv_v[pl.ds(o, 16)] = y
            dsqp_v[pl.ds(po, 16)] = y * y
            m_v[pl.ds(o, 16)] = xd * y

        pltpu.sync_copy(m_v, m_sh.at[pl.ds(noff, _NPT)])

    plsc.subcore_barrier()

    @pl.when(c == 0)
    def _phase3():
        pltpu.sync_copy(m_sh, m_full)

        @plsc.parallel_loop(0, _EPT // 16, unroll=5)
        def _msg_body(i):
            o = delta + i * 16
            ridx = ei_v[0, pl.ds(o, 16)]
            cidx = ei_v[1, pl.ds(o, 16)]
            mv = plsc.load_gather(m_full, [ridx])
            plsc.addupdate_scatter(s_loc, [cidx], mv)

        pltpu.sync_copy(s_loc, sh_all.at[s])

    plsc.subcore_barrier()

    @pl.when(c == 0)
    def _phase4():
        pltpu.sync_copy(sh_all.at[:, pl.ds(noff, _NPT)], red_v)

        @plsc.parallel_loop(0, _NPT // 16, unroll=4)
        def _e_body(j):
            o = j * 16
            tot = red_v[0, pl.ds(o, 16)]
            for t in range(1, _NS):
                tot = tot + red_v[t, pl.ds(o, 16)]
            po = (o // 64) * 128 + (o % 64)
            ep_v[pl.ds(po, 16)] = 64.0 * dinv_v[pl.ds(o, 16)] * tot

        poff = s * (_NPT * 2)
        pltpu.sync_copy(dsqp_v, out_hbm.at[pl.ds(poff, _NPT * 2)])
        pltpu.sync_copy(ep_v, out_hbm.at[pl.ds(_NPAD * 2 + poff, _NPT * 2)])
        pltpu.sync_copy(xdp_v, out_hbm.at[pl.ds(_NPAD * 4 + poff, _NPT * 2)])


_sc_edge = pl.kernel(
    _sc_body,
    out_type=jax.ShapeDtypeStruct((6 * _NPAD,), jnp.float32),
    mesh=plsc.VectorSubcoreMesh(core_axis_name="c", subcore_axis_name="s"),
    compiler_params=pltpu.CompilerParams(needs_layout_passes=False),
    scratch_types=[
        pltpu.VMEM((2, _EPTW), jnp.int32),
        pltpu.VMEM((_B, _XCN), jnp.float32),
        pltpu.VMEM((_NPT,), jnp.float32),
        pltpu.VMEM((_NPAD,), jnp.float32),
        pltpu.VMEM((_NPAD,), jnp.float32),
        pltpu.VMEM((_NPT,), jnp.float32),
        pltpu.VMEM((_NPAD,), jnp.float32),
        pltpu.VMEM((_NPT,), jnp.float32),
        pltpu.VMEM((2 * _NPT,), jnp.float32),
        pltpu.VMEM((2 * _NPT,), jnp.float32),
        pltpu.VMEM((2 * _NPT,), jnp.float32),
        pltpu.VMEM((_NS, _NPT), jnp.float32),
        pltpu.VMEM_SHARED((_NS, _NPAD), jnp.float32),
        pltpu.VMEM_SHARED((_NPAD,), jnp.float32),
    ],
)


def _tc_main_body(gw_ref, gb_ref, x_ref, w1_ref, o_ref):
    k = pl.program_id(0)
    gw = gw_ref[0, 0]
    gb = gb_ref[0]
    xb = x_ref[...]
    hb = jnp.maximum(gw * xb + gb, 0.0)
    col = k * _BK + lax.broadcasted_iota(jnp.int32, (_B, _BK), 1)
    x2b = jnp.where(col < _NEQ, hb, jnp.where(col < _XD, xb, 0.0))
    part = lax.dot_general(x2b, w1_ref[...], (((1,), (0,)), ((), ())),
                           preferred_element_type=jnp.float32)

    @pl.when(k == 0)
    def _():
        o_ref[...] = part

    @pl.when(k != 0)
    def _():
        o_ref[...] = o_ref[...] + part


_tc_main = pl.pallas_call(
    _tc_main_body,
    grid=(_NBLK,),
    in_specs=[
        pl.BlockSpec(memory_space=pltpu.SMEM),
        pl.BlockSpec(memory_space=pltpu.SMEM),
        pl.BlockSpec((_B, _BK), lambda k: (0, k)),
        pl.BlockSpec((_BK, _HID), lambda k: (k, 0)),
    ],
    out_specs=pl.BlockSpec((_B, _HID), lambda k: (0, 0)),
    out_shape=jax.ShapeDtypeStruct((_B, _HID), jnp.float32),
    compiler_params=pltpu.CompilerParams(dimension_semantics=("arbitrary",)),
)


def _tc_fix_body(gw_ref, gb_ref, sc_ref, w1_ref, acc_ref, b1_ref, w2_ref,
                 b2_ref, o_ref):
    gw = gw_ref[0, 0]
    gb = gb_ref[0]
    a = sc_ref[0]
    cc = sc_ref[1]
    xt = sc_ref[2]
    h_plain = jnp.maximum(gw * xt + gb, 0.0)
    h_corner = jnp.maximum(gw * (xt * a + cc) + gb, 0.0)
    delta_t = h_corner - h_plain
    fix = lax.dot_general(delta_t, w1_ref[:_CPAD, :], (((0,), (0,)), ((), ())),
                          preferred_element_type=jnp.float32)[:_B, :]
    h1 = jnp.maximum(acc_ref[...] + fix + b1_ref[...], 0.0)
    o_ref[...] = lax.dot_general(h1, w2_ref[...], (((1,), (0,)), ((), ())),
                                 preferred_element_type=jnp.float32) + b2_ref[...]


_tc_fix = pl.pallas_call(
    _tc_fix_body,
    grid=(1,),
    in_specs=[
        pl.BlockSpec(memory_space=pltpu.SMEM),
        pl.BlockSpec(memory_space=pltpu.SMEM),
        pl.BlockSpec((3, _CPAD, 128), lambda k: (0, 0, 0)),
        pl.BlockSpec((_CPAD, _HID), lambda k: (0, 0)),
        pl.BlockSpec((_B, _HID), lambda k: (0, 0)),
        pl.BlockSpec((1, _HID), lambda k: (0, 0)),
        pl.BlockSpec((_HID, _YD), lambda k: (0, 0)),
        pl.BlockSpec((1, _YD), lambda k: (0, 0)),
    ],
    out_specs=pl.BlockSpec((_B, _YD), lambda k: (0, 0)),
    out_shape=jax.ShapeDtypeStruct((_B, _YD), jnp.float32),
)


def kernel(x, edge_index, gcn_w, gcn_b, w1, b1, w2, b2):
    sc_out = _sc_edge(edge_index, x)
    acc = _tc_main(gcn_w, gcn_b, x, w1)
    sc3 = sc_out.reshape(3, _CPAD, 128)
    return _tc_fix(gcn_w, gcn_b, sc3, w1, acc, b1.reshape(1, _HID), w2,
                   b2.reshape(1, _YD))
